# Initial kernel scaffold; baseline (speedup 1.0000x reference)
import jax
import jax.numpy as jnp
from jax.experimental import pallas as pl


def kernel(input_batch, Wr, W1, b1, W2, b2):
    raise NotImplementedError("write your pallas kernel here")


# speedup vs baseline: 1.0650x; 1.0650x over previous
import functools

import jax
import jax.numpy as jnp
from jax.experimental import pallas as pl
from jax.experimental.pallas import tpu as pltpu

NUM_E = 8
TOPK = 2
T = 2048
D = 768
F = 3072
BF = 512
NF = F // BF


def _router_body(x_ref, wr_ref, gates_ref, loss_ref):
    x = x_ref[...]
    wr = wr_ref[...]
    logits = jnp.dot(x, wr, preferred_element_type=jnp.float32)
    m = jnp.max(logits, axis=-1, keepdims=True)
    ex = jnp.exp(logits - m)
    probs = ex / jnp.sum(ex, axis=-1, keepdims=True)
    lane = jax.lax.broadcasted_iota(jnp.int32, probs.shape, 1)
    p1 = jnp.max(probs, axis=-1, keepdims=True)
    is1 = (probs == p1)
    idx1 = jnp.min(jnp.where(is1, lane, NUM_E), axis=-1, keepdims=True)
    sel1 = lane == idx1
    probs2 = jnp.where(sel1, -1.0, probs)
    p2 = jnp.max(probs2, axis=-1, keepdims=True)
    is2 = (probs2 == p2)
    idx2 = jnp.min(jnp.where(is2, lane, NUM_E), axis=-1, keepdims=True)
    sel2 = lane == idx2
    gates = jnp.where(sel1, p1, 0.0) + jnp.where(sel2, p2, 0.0)
    gates_ref[...] = gates
    cnt = sel1.astype(jnp.float32) + sel2.astype(jnp.float32)
    frac = jnp.sum(cnt, axis=0) / (float(TOPK) * float(T))
    pmean = jnp.sum(probs, axis=0) / float(T)
    loss_ref[0, 0] = float(NUM_E) * jnp.sum(frac * pmean)


def _router(x, wr):
    gates, loss = pl.pallas_call(
        _router_body,
        out_shape=(
            jax.ShapeDtypeStruct((T, NUM_E), jnp.float32),
            jax.ShapeDtypeStruct((1, 1), jnp.float32),
        ),
        in_specs=[
            pl.BlockSpec((T, D), lambda: (0, 0)),
            pl.BlockSpec((D, NUM_E), lambda: (0, 0)),
        ],
        out_specs=(
            pl.BlockSpec((T, NUM_E), lambda: (0, 0)),
            pl.BlockSpec(memory_space=pltpu.SMEM),
        ),
    )(x, wr)
    return gates, loss


def _ffn_body(x_ref, w1_ref, b1_ref, w2_ref, b2_ref, g_ref, out_ref, acc_ref):
    e = pl.program_id(0)
    f = pl.program_id(1)
    x = x_ref[...]
    w1 = w1_ref[0]
    h = jnp.dot(x, w1, preferred_element_type=jnp.float32) + b1_ref[0]
    h = jax.nn.gelu(h)
    contrib = jnp.dot(h, w2_ref[0], preferred_element_type=jnp.float32)

    @pl.when(f == 0)
    def _init_acc():
        acc_ref[...] = contrib

    @pl.when(f != 0)
    def _add_acc():
        acc_ref[...] += contrib

    @pl.when(f == NF - 1)
    def _emit():
        g = g_ref[...]
        lane = jax.lax.broadcasted_iota(jnp.int32, g.shape, 1)
        gcol = jnp.sum(jnp.where(lane == e, g, 0.0), axis=1, keepdims=True)
        o = (acc_ref[...] + b2_ref[0]) * gcol

        @pl.when(e == 0)
        def _():
            out_ref[...] = o

        @pl.when(e != 0)
        def _():
            out_ref[...] += o


def _ffn(x, w1, b1, w2, b2, gates):
    return pl.pallas_call(
        _ffn_body,
        grid=(NUM_E, NF),
        out_shape=jax.ShapeDtypeStruct((T, D), jnp.float32),
        in_specs=[
            pl.BlockSpec((T, D), lambda e, f: (0, 0)),
            pl.BlockSpec((1, D, BF), lambda e, f: (e, 0, f)),
            pl.BlockSpec((1, 1, BF), lambda e, f: (e, 0, f)),
            pl.BlockSpec((1, BF, D), lambda e, f: (e, f, 0)),
            pl.BlockSpec((1, 1, D), lambda e, f: (e, 0, 0)),
            pl.BlockSpec((T, NUM_E), lambda e, f: (0, 0)),
        ],
        out_specs=pl.BlockSpec((T, D), lambda e, f: (0, 0)),
        scratch_shapes=[pltpu.VMEM((T, D), jnp.float32)],
    )(x, w1, b1.reshape(NUM_E, 1, F), w2, b2.reshape(NUM_E, 1, D), gates)


def kernel(input_batch, Wr, W1, b1, W2, b2):
    B, S, Dm = input_batch.shape
    x = input_batch.reshape(B * S, Dm)
    gates, loss = _router(x, Wr)
    out = _ffn(x, W1, b1, W2, b2, gates)
    return out.reshape(B, S, Dm), loss[0, 0]
